# Initial kernel scaffold; baseline (speedup 1.0000x reference)
import jax
import jax.numpy as jnp
from jax.experimental import pallas as pl


def kernel(hidden_states, gate_w, ws, w2s):
    raise NotImplementedError("write your pallas kernel here")


# speedup vs baseline: 1.2314x; 1.2314x over previous
import functools

import jax
import jax.numpy as jnp
from jax.experimental import pallas as pl
from jax.experimental.pallas import tpu as pltpu

TOP_K = 2


def _moe_body(x_ref, gw_ref, wg_ref, wu_ref, w2_ref, o_ref, *, n_experts):
    e = pl.program_id(0)
    f = pl.program_id(1)
    x = x_ref[...]
    T = x.shape[0]

    logits = jnp.dot(x, gw_ref[...], preferred_element_type=jnp.float32)
    cols = jax.lax.broadcasted_iota(jnp.int32, logits.shape, 1)
    l1 = jnp.max(logits, axis=-1)
    m1 = logits == l1[:, None]
    i1 = jnp.min(jnp.where(m1, cols, n_experts), axis=-1)
    first1 = cols == i1[:, None]
    masked = jnp.where(first1, -jnp.inf, logits)
    l2 = jnp.max(masked, axis=-1)
    m2 = masked == l2[:, None]
    i2 = jnp.min(jnp.where(m2, cols, n_experts), axis=-1)
    w1 = jax.nn.sigmoid(l1 - l2)
    w2 = 1.0 - w1
    we = jnp.where(i1 == e, w1, jnp.where(i2 == e, w2, 0.0))

    g = jnp.dot(x, wg_ref[0], preferred_element_type=jnp.float32)
    u = jnp.dot(x, wu_ref[0], preferred_element_type=jnp.float32)
    h = (g * jax.nn.sigmoid(g)) * u
    y = jnp.dot(h, w2_ref[0], preferred_element_type=jnp.float32)
    contrib = y * we[:, None]

    @pl.when((e == 0) & (f == 0))
    def _init():
        o_ref[...] = contrib

    @pl.when((e > 0) | (f > 0))
    def _acc():
        o_ref[...] += contrib


def kernel(hidden_states, gate_w, ws, w2s):
    B, S, D = hidden_states.shape
    T = B * S
    E = gate_w.shape[1]
    F = ws.shape[-1] // 2
    FC = 512
    NF = F // FC

    x = hidden_states.reshape(T, D)
    w_gate = ws[:, :, :F]
    w_up = ws[:, :, F:]

    out = pl.pallas_call(
        functools.partial(_moe_body, n_experts=E),
        grid=(E, NF),
        in_specs=[
            pl.BlockSpec((T, D), lambda e, f: (0, 0)),
            pl.BlockSpec((D, E), lambda e, f: (0, 0)),
            pl.BlockSpec((1, D, FC), lambda e, f: (e, 0, f)),
            pl.BlockSpec((1, D, FC), lambda e, f: (e, 0, f)),
            pl.BlockSpec((1, FC, D), lambda e, f: (e, f, 0)),
        ],
        out_specs=pl.BlockSpec((T, D), lambda e, f: (0, 0)),
        out_shape=jax.ShapeDtypeStruct((T, D), jnp.float32),
        compiler_params=pltpu.CompilerParams(
            dimension_semantics=("arbitrary", "arbitrary"),
        ),
    )(x, gate_w, w_gate, w_up, w2s)

    return out.reshape(B, S, D).astype(hidden_states.dtype)


# speedup vs baseline: 1.2373x; 1.0048x over previous
import functools

import jax
import jax.numpy as jnp
from jax.experimental import pallas as pl
from jax.experimental.pallas import tpu as pltpu

TOP_K = 2


def _moe_body(x_ref, gw_ref, wg_ref, wu_ref, w2_ref, o_ref, *, n_experts):
    e = pl.program_id(0)
    f = pl.program_id(1)
    x = x_ref[...]
    T = x.shape[0]

    logits = jnp.dot(x, gw_ref[...], preferred_element_type=jnp.float32)
    cols = jax.lax.broadcasted_iota(jnp.int32, logits.shape, 1)
    l1 = jnp.max(logits, axis=-1)
    m1 = logits == l1[:, None]
    i1 = jnp.min(jnp.where(m1, cols, n_experts), axis=-1)
    first1 = cols == i1[:, None]
    masked = jnp.where(first1, -jnp.inf, logits)
    l2 = jnp.max(masked, axis=-1)
    m2 = masked == l2[:, None]
    i2 = jnp.min(jnp.where(m2, cols, n_experts), axis=-1)
    w1 = jax.nn.sigmoid(l1 - l2)
    w2 = 1.0 - w1
    we = jnp.where(i1 == e, w1, jnp.where(i2 == e, w2, 0.0))

    xb = x.astype(jnp.bfloat16)
    g = jnp.dot(xb, wg_ref[0].astype(jnp.bfloat16),
                preferred_element_type=jnp.float32)
    u = jnp.dot(xb, wu_ref[0].astype(jnp.bfloat16),
                preferred_element_type=jnp.float32)
    h = (g * jax.nn.sigmoid(g)) * u
    y = jnp.dot(h.astype(jnp.bfloat16), w2_ref[0].astype(jnp.bfloat16),
                preferred_element_type=jnp.float32)
    contrib = y * we[:, None]

    @pl.when((e == 0) & (f == 0))
    def _init():
        o_ref[...] = contrib

    @pl.when((e > 0) | (f > 0))
    def _acc():
        o_ref[...] += contrib


def kernel(hidden_states, gate_w, ws, w2s):
    B, S, D = hidden_states.shape
    T = B * S
    E = gate_w.shape[1]
    F = ws.shape[-1] // 2
    FC = 512
    NF = F // FC

    x = hidden_states.reshape(T, D)
    w_gate = ws[:, :, :F]
    w_up = ws[:, :, F:]

    out = pl.pallas_call(
        functools.partial(_moe_body, n_experts=E),
        grid=(E, NF),
        in_specs=[
            pl.BlockSpec((T, D), lambda e, f: (0, 0)),
            pl.BlockSpec((D, E), lambda e, f: (0, 0)),
            pl.BlockSpec((1, D, FC), lambda e, f: (e, 0, f)),
            pl.BlockSpec((1, D, FC), lambda e, f: (e, 0, f)),
            pl.BlockSpec((1, FC, D), lambda e, f: (e, f, 0)),
        ],
        out_specs=pl.BlockSpec((T, D), lambda e, f: (0, 0)),
        out_shape=jax.ShapeDtypeStruct((T, D), jnp.float32),
        compiler_params=pltpu.CompilerParams(
            dimension_semantics=("arbitrary", "arbitrary"),
        ),
    )(x, gate_w, w_gate, w_up, w2s)

    return out.reshape(B, S, D).astype(hidden_states.dtype)


# speedup vs baseline: 1.2575x; 1.0163x over previous
import functools

import jax
import jax.numpy as jnp
from jax import lax
from jax.experimental import pallas as pl
from jax.experimental.pallas import tpu as pltpu
from jax.experimental.pallas import tpu_sc as plsc

TOP_K = 2
BS = 128
EIDS_LEN = 64


def _router_body(x_ref, gw_ref, posa_ref, posb_ref, w1_ref, w2_ref, eids_ref,
                 *, n_experts, n_blocks):
    x = x_ref[...]
    T = x.shape[0]
    E = n_experts

    logits = jnp.dot(x, gw_ref[...], preferred_element_type=jnp.float32)
    cols = lax.broadcasted_iota(jnp.int32, logits.shape, 1)
    l1 = jnp.max(logits, axis=-1)
    m1 = logits == l1[:, None]
    i1 = jnp.min(jnp.where(m1, cols, E), axis=-1)
    first1 = cols == i1[:, None]
    masked = jnp.where(first1, -jnp.inf, logits)
    l2 = jnp.max(masked, axis=-1)
    m2 = masked == l2[:, None]
    i2 = jnp.min(jnp.where(m2, cols, E), axis=-1)
    w1 = jax.nn.sigmoid(l1 - l2)
    w1_ref[...] = w1
    w2_ref[...] = 1.0 - w1

    o1 = first1.astype(jnp.int32)
    o2 = (cols == i2[:, None]).astype(jnp.int32)
    s = o1 + o2
    c = s
    sh = 1
    while sh < T:
        c = c + jnp.concatenate(
            [jnp.zeros((sh, E), jnp.int32), c[: T - sh]], axis=0)
        sh *= 2
    ecs = c - s
    counts = c[T - 1:T, :]
    nb = (counts + (BS - 1)) // BS
    bc = nb
    sh = 1
    while sh < E:
        bc = bc + jnp.concatenate(
            [jnp.zeros((1, sh), jnp.int32), bc[:, : E - sh]], axis=1)
        sh *= 2
    bc_excl = bc - nb
    nbtot = bc[:, E - 1:E]
    base = bc_excl * BS

    dest = base + ecs
    posa_ref[...] = jnp.sum(o1 * dest, axis=1)
    posb_ref[...] = jnp.sum(o2 * dest, axis=1)

    bvec = lax.broadcasted_iota(jnp.int32, (1, EIDS_LEN), 1)
    bclamp = jnp.minimum(bvec, nbtot - 1)
    ge = (bc_excl[:, :, None] <= bclamp[:, None, :]).astype(jnp.int32)
    eids = jnp.sum(ge, axis=1) - 1
    eids_ref[...] = jnp.where(bvec == n_blocks, nbtot, eids)


def _dispatch_body(posa, posb, w1h, w2h, x_hbm, xs_hbm, sw_hbm,
                   pa_v, pb_v, wa_v, wb_v, st_v, sw_v, rows_v, sem,
                   *, T, P_pad, rows_per_tile, chunk):
    nc = 2
    wid = lax.axis_index("s") * nc + lax.axis_index("c")
    pltpu.sync_copy(posa, pa_v)
    pltpu.sync_copy(posb, pb_v)
    pltpu.sync_copy(w1h, wa_v)
    pltpu.sync_copy(w2h, wb_v)

    zi = jnp.zeros((16,), jnp.int32)
    zf = jnp.zeros((16,), jnp.float32)

    def init(i, _):
        st_v[pl.ds(i * 16, 16)] = zi
        sw_v[pl.ds(i * 16, 16)] = zf
        return 0

    lax.fori_loop(0, P_pad // 16, init, 0)

    lane = lax.iota(jnp.int32, 16)

    def scat(p, _):
        tvec = p * 16 + lane
        plsc.store_scatter(st_v, [pa_v[pl.ds(p * 16, 16)]], tvec)
        plsc.store_scatter(sw_v, [pa_v[pl.ds(p * 16, 16)]],
                           wa_v[pl.ds(p * 16, 16)])
        plsc.store_scatter(st_v, [pb_v[pl.ds(p * 16, 16)]], tvec)
        plsc.store_scatter(sw_v, [pb_v[pl.ds(p * 16, 16)]],
                           wb_v[pl.ds(p * 16, 16)])
        return 0

    lax.fori_loop(0, T // 16, scat, 0)

    base = wid * rows_per_tile
    pltpu.sync_copy(sw_v.at[pl.ds(base, rows_per_tile)],
                    sw_hbm.at[pl.ds(base, rows_per_tile)])
    for c in range(rows_per_tile // chunk):
        off = base + c * chunk
        pltpu.async_copy(x_hbm.at[st_v.at[pl.ds(off, chunk)]], rows_v,
                         sem).wait()
        pltpu.sync_copy(rows_v, xs_hbm.at[pl.ds(off, chunk)])


def _gemm_body(eids_ref, xs_ref, wg_ref, wu_ref, w2_ref, sw_ref, ys_ref,
               *, n_blocks):
    b = pl.program_id(0)
    nbtot = eids_ref[n_blocks]

    @pl.when(b < nbtot)
    def _run():
        xb = xs_ref[...]
        g = jnp.dot(xb, wg_ref[0], preferred_element_type=jnp.float32)
        u = jnp.dot(xb, wu_ref[0], preferred_element_type=jnp.float32)
        h = (g * jax.nn.sigmoid(g)) * u
        y = jnp.dot(h, w2_ref[0], preferred_element_type=jnp.float32)
        ys_ref[...] = y * sw_ref[0, 0][:, None]


def _combine_body(posa, posb, ys_hbm, out_hbm,
                  pa_v, pb_v, ra_v, rb_v, ro_v, sem,
                  *, tok_per_tile, chunk, D):
    nc = 2
    wid = lax.axis_index("s") * nc + lax.axis_index("c")
    base = wid * tok_per_tile
    pltpu.sync_copy(posa.at[pl.ds(base, tok_per_tile)], pa_v)
    pltpu.sync_copy(posb.at[pl.ds(base, tok_per_tile)], pb_v)
    nvec = D // 16
    for c in range(tok_per_tile // chunk):
        pltpu.async_copy(ys_hbm.at[pa_v.at[pl.ds(c * chunk, chunk)]], ra_v,
                         sem).wait()
        pltpu.async_copy(ys_hbm.at[pb_v.at[pl.ds(c * chunk, chunk)]], rb_v,
                         sem).wait()

        def row(i, _):
            def col(j, _):
                ro_v[i, pl.ds(j * 16, 16)] = (
                    ra_v[i, pl.ds(j * 16, 16)] + rb_v[i, pl.ds(j * 16, 16)])
                return 0

            lax.fori_loop(0, nvec, col, 0, unroll=8)
            return 0

        lax.fori_loop(0, chunk, row, 0)
        pltpu.sync_copy(ro_v, out_hbm.at[pl.ds(base + c * chunk, chunk)])


def kernel(hidden_states, gate_w, ws, w2s):
    B, S, D = hidden_states.shape
    T = B * S
    E = gate_w.shape[1]
    F = ws.shape[-1] // 2
    NB = T * TOP_K // BS + E
    P_pad = NB * BS

    x = hidden_states.reshape(T, D)

    posa, posb, w1, w2, eids2d = pl.pallas_call(
        functools.partial(_router_body, n_experts=E, n_blocks=NB),
        grid=(1,),
        in_specs=[
            pl.BlockSpec((T, D), lambda i: (0, 0)),
            pl.BlockSpec((D, E), lambda i: (0, 0)),
        ],
        out_specs=[
            pl.BlockSpec((T,), lambda i: (0,)),
            pl.BlockSpec((T,), lambda i: (0,)),
            pl.BlockSpec((T,), lambda i: (0,)),
            pl.BlockSpec((T,), lambda i: (0,)),
            pl.BlockSpec((1, EIDS_LEN), lambda i: (0, 0)),
        ],
        out_shape=[
            jax.ShapeDtypeStruct((T,), jnp.int32),
            jax.ShapeDtypeStruct((T,), jnp.int32),
            jax.ShapeDtypeStruct((T,), jnp.float32),
            jax.ShapeDtypeStruct((T,), jnp.float32),
            jax.ShapeDtypeStruct((1, EIDS_LEN), jnp.int32),
        ],
    )(x, gate_w)

    NW = 32
    rpt = P_pad // NW
    chunk = rpt // 2
    mesh = plsc.VectorSubcoreMesh(core_axis_name="c", subcore_axis_name="s")
    xs, sw = pl.kernel(
        functools.partial(_dispatch_body, T=T, P_pad=P_pad,
                          rows_per_tile=rpt, chunk=chunk),
        out_type=[
            jax.ShapeDtypeStruct((P_pad, D), jnp.float32),
            jax.ShapeDtypeStruct((P_pad,), jnp.float32),
        ],
        mesh=mesh,
        scratch_types=[
            pltpu.VMEM((T,), jnp.int32),
            pltpu.VMEM((T,), jnp.int32),
            pltpu.VMEM((T,), jnp.float32),
            pltpu.VMEM((T,), jnp.float32),
            pltpu.VMEM((P_pad,), jnp.int32),
            pltpu.VMEM((P_pad,), jnp.float32),
            pltpu.VMEM((chunk, D), jnp.float32),
            pltpu.SemaphoreType.DMA,
        ],
        compiler_params=pltpu.CompilerParams(needs_layout_passes=False),
    )(posa, posb, w1, w2, x)

    eids = eids2d.reshape(EIDS_LEN)
    sw3 = sw.reshape(NB, 1, BS)
    w_gate = ws[:, :, :F]
    w_up = ws[:, :, F:]

    ys = pl.pallas_call(
        functools.partial(_gemm_body, n_blocks=NB),
        grid_spec=pltpu.PrefetchScalarGridSpec(
            num_scalar_prefetch=1,
            grid=(NB,),
            in_specs=[
                pl.BlockSpec((BS, D), lambda b, e: (b, 0)),
                pl.BlockSpec((1, D, F), lambda b, e: (e[b], 0, 0)),
                pl.BlockSpec((1, D, F), lambda b, e: (e[b], 0, 0)),
                pl.BlockSpec((1, F, D), lambda b, e: (e[b], 0, 0)),
                pl.BlockSpec((1, 1, BS), lambda b, e: (b, 0, 0)),
            ],
            out_specs=pl.BlockSpec((BS, D), lambda b, e: (b, 0)),
        ),
        out_shape=jax.ShapeDtypeStruct((P_pad, D), jnp.float32),
        compiler_params=pltpu.CompilerParams(
            dimension_semantics=("arbitrary",),
        ),
    )(eids, xs, w_gate, w_up, w2s, sw3)

    tpt = T // NW
    cchunk = tpt // 2
    out = pl.kernel(
        functools.partial(_combine_body, tok_per_tile=tpt, chunk=cchunk, D=D),
        out_type=jax.ShapeDtypeStruct((T, D), jnp.float32),
        mesh=mesh,
        scratch_types=[
            pltpu.VMEM((tpt,), jnp.int32),
            pltpu.VMEM((tpt,), jnp.int32),
            pltpu.VMEM((cchunk, D), jnp.float32),
            pltpu.VMEM((cchunk, D), jnp.float32),
            pltpu.VMEM((cchunk, D), jnp.float32),
            pltpu.SemaphoreType.DMA,
        ],
    )(posa, posb, ys)

    return out.reshape(B, S, D).astype(hidden_states.dtype)


# speedup vs baseline: 1.4620x; 1.1626x over previous
import functools

import jax
import jax.numpy as jnp
from jax import lax
from jax.experimental import pallas as pl
from jax.experimental.pallas import tpu as pltpu
from jax.experimental.pallas import tpu_sc as plsc

TOP_K = 2
BS = 128
EIDS_LEN = 64


def _router_body(x_ref, gw_ref, posa_ref, posb_ref, w1_ref, w2_ref, eids_ref,
                 *, n_experts, n_blocks):
    x = x_ref[...]
    T = x.shape[0]
    E = n_experts

    logits = jnp.dot(x, gw_ref[...], preferred_element_type=jnp.float32)
    cols = lax.broadcasted_iota(jnp.int32, logits.shape, 1)
    l1 = jnp.max(logits, axis=-1)
    m1 = logits == l1[:, None]
    i1 = jnp.min(jnp.where(m1, cols, E), axis=-1)
    first1 = cols == i1[:, None]
    masked = jnp.where(first1, -jnp.inf, logits)
    l2 = jnp.max(masked, axis=-1)
    m2 = masked == l2[:, None]
    i2 = jnp.min(jnp.where(m2, cols, E), axis=-1)
    w1 = jax.nn.sigmoid(l1 - l2)
    w1_ref[...] = w1
    w2_ref[...] = 1.0 - w1

    o1 = first1.astype(jnp.int32)
    o2 = (cols == i2[:, None]).astype(jnp.int32)
    s = o1 + o2
    c = s
    sh = 1
    while sh < T:
        c = c + jnp.concatenate(
            [jnp.zeros((sh, E), jnp.int32), c[: T - sh]], axis=0)
        sh *= 2
    ecs = c - s
    counts = c[T - 1:T, :]
    nb = (counts + (BS - 1)) // BS
    bc = nb
    sh = 1
    while sh < E:
        bc = bc + jnp.concatenate(
            [jnp.zeros((1, sh), jnp.int32), bc[:, : E - sh]], axis=1)
        sh *= 2
    bc_excl = bc - nb
    nbtot = bc[:, E - 1:E]
    base = bc_excl * BS

    dest = base + ecs
    posa_ref[...] = jnp.sum(o1 * dest, axis=1)
    posb_ref[...] = jnp.sum(o2 * dest, axis=1)

    bvec = lax.broadcasted_iota(jnp.int32, (1, EIDS_LEN), 1)
    bclamp = jnp.minimum(bvec, nbtot - 1)
    ge = (bc_excl[:, :, None] <= bclamp[:, None, :]).astype(jnp.int32)
    eids = jnp.sum(ge, axis=1) - 1
    eids_ref[...] = jnp.where(bvec == n_blocks, nbtot, eids)


def _dispatch_body(posa, posb, w1h, w2h, x_hbm, xs_hbm, sw_hbm,
                   pa_v, pb_v, wa_v, wb_v, rows_v, sem,
                   *, tok_per_tile):
    nc = 2
    wid = lax.axis_index("s") * nc + lax.axis_index("c")
    base = wid * tok_per_tile
    pltpu.sync_copy(posa.at[pl.ds(base, tok_per_tile)], pa_v)
    pltpu.sync_copy(posb.at[pl.ds(base, tok_per_tile)], pb_v)
    pltpu.sync_copy(w1h.at[pl.ds(base, tok_per_tile)], wa_v)
    pltpu.sync_copy(w2h.at[pl.ds(base, tok_per_tile)], wb_v)
    pltpu.sync_copy(x_hbm.at[pl.ds(base, tok_per_tile)], rows_v)
    ca = pltpu.async_copy(rows_v, xs_hbm.at[pa_v], sem)
    cb = pltpu.async_copy(rows_v, xs_hbm.at[pb_v], sem)
    cc = pltpu.async_copy(wa_v, sw_hbm.at[pa_v], sem)
    cd = pltpu.async_copy(wb_v, sw_hbm.at[pb_v], sem)
    ca.wait()
    cb.wait()
    cc.wait()
    cd.wait()


def _gemm_body(eids_ref, xs_ref, wg_ref, wu_ref, w2_ref, sw_ref, ys_ref,
               *, n_blocks):
    b = pl.program_id(0)
    nbtot = eids_ref[n_blocks]

    @pl.when(b < nbtot)
    def _run():
        xb = xs_ref[...]
        g = jnp.dot(xb, wg_ref[0], preferred_element_type=jnp.float32)
        u = jnp.dot(xb, wu_ref[0], preferred_element_type=jnp.float32)
        h = (g * jax.nn.sigmoid(g)) * u
        y = jnp.dot(h, w2_ref[0], preferred_element_type=jnp.float32)
        ys_ref[...] = y * sw_ref[0, 0][:, None]


def _combine_body(posa, posb, ys_hbm, out_hbm,
                  pa_v, pb_v, ra_v, rb_v, ro_v, sem,
                  *, tok_per_tile, chunk, D):
    nc = 2
    wid = lax.axis_index("s") * nc + lax.axis_index("c")
    base = wid * tok_per_tile
    pltpu.sync_copy(posa.at[pl.ds(base, tok_per_tile)], pa_v)
    pltpu.sync_copy(posb.at[pl.ds(base, tok_per_tile)], pb_v)
    nvec = D // 16
    for c in range(tok_per_tile // chunk):
        pltpu.async_copy(ys_hbm.at[pa_v.at[pl.ds(c * chunk, chunk)]], ra_v,
                         sem).wait()
        pltpu.async_copy(ys_hbm.at[pb_v.at[pl.ds(c * chunk, chunk)]], rb_v,
                         sem).wait()

        def row(i, _):
            def col(j, _):
                ro_v[i, pl.ds(j * 16, 16)] = (
                    ra_v[i, pl.ds(j * 16, 16)] + rb_v[i, pl.ds(j * 16, 16)])
                return 0

            lax.fori_loop(0, nvec, col, 0, unroll=8)
            return 0

        lax.fori_loop(0, chunk, row, 0)
        pltpu.sync_copy(ro_v, out_hbm.at[pl.ds(base + c * chunk, chunk)])


def kernel(hidden_states, gate_w, ws, w2s):
    B, S, D = hidden_states.shape
    T = B * S
    E = gate_w.shape[1]
    F = ws.shape[-1] // 2
    NB = T * TOP_K // BS + E
    P_pad = NB * BS

    x = hidden_states.reshape(T, D)

    posa, posb, w1, w2, eids2d = pl.pallas_call(
        functools.partial(_router_body, n_experts=E, n_blocks=NB),
        grid=(1,),
        in_specs=[
            pl.BlockSpec((T, D), lambda i: (0, 0)),
            pl.BlockSpec((D, E), lambda i: (0, 0)),
        ],
        out_specs=[
            pl.BlockSpec((T,), lambda i: (0,)),
            pl.BlockSpec((T,), lambda i: (0,)),
            pl.BlockSpec((T,), lambda i: (0,)),
            pl.BlockSpec((T,), lambda i: (0,)),
            pl.BlockSpec((1, EIDS_LEN), lambda i: (0, 0)),
        ],
        out_shape=[
            jax.ShapeDtypeStruct((T,), jnp.int32),
            jax.ShapeDtypeStruct((T,), jnp.int32),
            jax.ShapeDtypeStruct((T,), jnp.float32),
            jax.ShapeDtypeStruct((T,), jnp.float32),
            jax.ShapeDtypeStruct((1, EIDS_LEN), jnp.int32),
        ],
    )(x, gate_w)

    NW = 32
    tpt_d = T // NW
    mesh = plsc.VectorSubcoreMesh(core_axis_name="c", subcore_axis_name="s")
    xs, sw = pl.kernel(
        functools.partial(_dispatch_body, tok_per_tile=tpt_d),
        out_type=[
            jax.ShapeDtypeStruct((P_pad, D), jnp.float32),
            jax.ShapeDtypeStruct((P_pad,), jnp.float32),
        ],
        mesh=mesh,
        scratch_types=[
            pltpu.VMEM((tpt_d,), jnp.int32),
            pltpu.VMEM((tpt_d,), jnp.int32),
            pltpu.VMEM((tpt_d,), jnp.float32),
            pltpu.VMEM((tpt_d,), jnp.float32),
            pltpu.VMEM((tpt_d, D), jnp.float32),
            pltpu.SemaphoreType.DMA,
        ],
        compiler_params=pltpu.CompilerParams(needs_layout_passes=False),
    )(posa, posb, w1, w2, x)

    eids = eids2d.reshape(EIDS_LEN)
    sw3 = sw.reshape(NB, 1, BS)
    w_gate = ws[:, :, :F]
    w_up = ws[:, :, F:]

    ys = pl.pallas_call(
        functools.partial(_gemm_body, n_blocks=NB),
        grid_spec=pltpu.PrefetchScalarGridSpec(
            num_scalar_prefetch=1,
            grid=(NB,),
            in_specs=[
                pl.BlockSpec((BS, D), lambda b, e: (b, 0)),
                pl.BlockSpec((1, D, F), lambda b, e: (e[b], 0, 0)),
                pl.BlockSpec((1, D, F), lambda b, e: (e[b], 0, 0)),
                pl.BlockSpec((1, F, D), lambda b, e: (e[b], 0, 0)),
                pl.BlockSpec((1, 1, BS), lambda b, e: (b, 0, 0)),
            ],
            out_specs=pl.BlockSpec((BS, D), lambda b, e: (b, 0)),
        ),
        out_shape=jax.ShapeDtypeStruct((P_pad, D), jnp.float32),
        compiler_params=pltpu.CompilerParams(
            dimension_semantics=("arbitrary",),
        ),
    )(eids, xs, w_gate, w_up, w2s, sw3)

    tpt = T // NW
    cchunk = tpt // 2
    out = pl.kernel(
        functools.partial(_combine_body, tok_per_tile=tpt, chunk=cchunk, D=D),
        out_type=jax.ShapeDtypeStruct((T, D), jnp.float32),
        mesh=mesh,
        scratch_types=[
            pltpu.VMEM((tpt,), jnp.int32),
            pltpu.VMEM((tpt,), jnp.int32),
            pltpu.VMEM((cchunk, D), jnp.float32),
            pltpu.VMEM((cchunk, D), jnp.float32),
            pltpu.VMEM((cchunk, D), jnp.float32),
            pltpu.SemaphoreType.DMA,
        ],
    )(posa, posb, ys)

    return out.reshape(B, S, D).astype(hidden_states.dtype)
